# Initial kernel scaffold; baseline (speedup 1.0000x reference)
import jax
import jax.numpy as jnp
from jax.experimental import pallas as pl


def kernel(x, x_len, pe):
    raise NotImplementedError("write your pallas kernel here")


# speedup vs baseline: 2.0312x; 2.0312x over previous
import functools

import jax
import jax.numpy as jnp
from jax import lax
from jax.experimental import pallas as pl
from jax.experimental.pallas import tpu as pltpu
from jax.experimental.pallas import tpu_sc as plsc

BATCH = 4096
SEQ = 200
D = 64
MAXL = 250
QROWS = 456
NTILES = 32
RPT = BATCH // NTILES
LANES = 16


def _body(x_hbm, len_hbm, q_hbm, out_hbm,
          q_v, len_v, xb0, xb1, ob0, ob1,
          in_s0, in_s1, out_s0, out_s1):
    wid = lax.axis_index("s") * 2 + lax.axis_index("c")
    base = wid * RPT

    pltpu.sync_copy(q_hbm, q_v)
    pltpu.sync_copy(len_hbm.at[pl.ds(base, RPT)], len_v)

    pltpu.make_async_copy(x_hbm.at[base], xb0, in_s0).start()
    pltpu.make_async_copy(x_hbm.at[base + 1], xb1, in_s1).start()

    xbufs = (xb0, xb1)
    obufs = (ob0, ob1)
    in_sems = (in_s0, in_s1)
    out_sems = (out_s0, out_s1)

    def group(g, _):
        svec = (MAXL - 1) - jnp.minimum(len_v[pl.ds(g * LANES, LANES)], MAXL - 1)
        for lane in range(LANES):
            i = g * LANES + lane
            b = lane % 2
            xb, ob = xbufs[b], obufs[b]
            pltpu.make_async_copy(x_hbm.at[base], xb, in_sems[b]).wait()
            s = svec[lane]

            @pl.when(i >= 2)
            def _():
                pltpu.make_async_copy(ob, out_hbm.at[base], out_sems[b]).wait()

            def row(j, _):
                for c in range(D // LANES):
                    col = pl.ds(c * LANES, LANES)
                    ob[j, col] = xb[j, col] + q_v[s + j, col]
                return 0

            lax.fori_loop(0, SEQ, row, 0, unroll=2)

            pltpu.make_async_copy(ob, out_hbm.at[base + i], out_sems[b]).start()

            @pl.when(i + 2 < RPT)
            def _():
                pltpu.make_async_copy(x_hbm.at[base + i + 2], xb, in_sems[b]).start()
        return 0

    lax.fori_loop(0, RPT // LANES, group, 0)

    pltpu.make_async_copy(ob0, out_hbm.at[base], out_s0).wait()
    pltpu.make_async_copy(ob1, out_hbm.at[base], out_s1).wait()


@functools.partial(jax.jit, static_argnums=())
def kernel(x, x_len, pe):
    q = jnp.concatenate(
        [jnp.flip(pe[:MAXL], axis=0),
         jnp.broadcast_to(pe[0:1], (QROWS - MAXL, D))], axis=0)
    xl = x_len.astype(jnp.int32)

    run = pl.kernel(
        _body,
        out_type=jax.ShapeDtypeStruct((BATCH, SEQ, D), jnp.float32),
        mesh=plsc.VectorSubcoreMesh(core_axis_name="c", subcore_axis_name="s"),
        compiler_params=pltpu.CompilerParams(use_tc_tiling_on_sc=False),
        scratch_types=[
            pltpu.VMEM((QROWS, D), jnp.float32),
            pltpu.VMEM((RPT,), jnp.int32),
            pltpu.VMEM((SEQ, D), jnp.float32),
            pltpu.VMEM((SEQ, D), jnp.float32),
            pltpu.VMEM((SEQ, D), jnp.float32),
            pltpu.VMEM((SEQ, D), jnp.float32),
            pltpu.SemaphoreType.DMA,
            pltpu.SemaphoreType.DMA,
            pltpu.SemaphoreType.DMA,
            pltpu.SemaphoreType.DMA,
        ],
    )
    return run(x, xl, q)


# speedup vs baseline: 2.7304x; 1.3443x over previous
import functools

import jax
import jax.numpy as jnp
from jax import lax
from jax.experimental import pallas as pl
from jax.experimental.pallas import tpu as pltpu
from jax.experimental.pallas import tpu_sc as plsc

BATCH = 4096
SEQ = 200
D = 64
MAXL = 250
QROWS = 456
NTILES = 32
RPT = BATCH // NTILES
LANES = 16


NIB = 4
NOB = 2


def _body(x_hbm, len_hbm, q_hbm, out_hbm,
          q_v, len_v, *bufs_and_sems):
    xbufs = bufs_and_sems[0:NIB]
    obufs = bufs_and_sems[NIB:NIB + NOB]
    in_sems = bufs_and_sems[NIB + NOB:2 * NIB + NOB]
    out_sems = bufs_and_sems[2 * NIB + NOB:2 * NIB + 2 * NOB]

    wid = lax.axis_index("s") * 2 + lax.axis_index("c")
    base = wid * RPT

    pltpu.sync_copy(q_hbm, q_v)
    pltpu.sync_copy(len_hbm.at[pl.ds(base, RPT)], len_v)

    for b in range(NIB):
        pltpu.make_async_copy(x_hbm.at[base + b], xbufs[b], in_sems[b]).start()

    def group(g, _):
        svec = (MAXL - 1) - jnp.minimum(len_v[pl.ds(g * LANES, LANES)], MAXL - 1)
        for lane in range(LANES):
            i = g * LANES + lane
            b = lane % NIB
            bo = lane % NOB
            xb, ob = xbufs[b], obufs[bo]
            pltpu.make_async_copy(x_hbm.at[base], xb, in_sems[b]).wait()
            s = svec[lane]

            @pl.when(i >= NOB)
            def _():
                pltpu.make_async_copy(ob, out_hbm.at[base], out_sems[bo]).wait()

            RU = 4

            def row(t, _):
                jb = t * RU
                chunks = []
                for dj in range(RU):
                    for c in range(D // LANES):
                        col = pl.ds(c * LANES, LANES)
                        chunks.append((jb + dj, col, xb[jb + dj, col]))
                qvals = [q_v[s + j, col] for (j, col, _) in chunks]
                for (j, col, xv), qv in zip(chunks, qvals):
                    ob[j, col] = xv + qv
                return 0

            lax.fori_loop(0, SEQ // RU, row, 0)

            pltpu.make_async_copy(ob, out_hbm.at[base + i], out_sems[bo]).start()

            @pl.when(i + NIB < RPT)
            def _():
                pltpu.make_async_copy(x_hbm.at[base + i + NIB], xb, in_sems[b]).start()
        return 0

    lax.fori_loop(0, RPT // LANES, group, 0)

    for bo in range(NOB):
        pltpu.make_async_copy(obufs[bo], out_hbm.at[base], out_sems[bo]).wait()


@functools.partial(jax.jit, static_argnums=())
def kernel(x, x_len, pe):
    q = jnp.concatenate(
        [jnp.flip(pe[:MAXL], axis=0),
         jnp.broadcast_to(pe[0:1], (QROWS - MAXL, D))], axis=0)
    xl = x_len.astype(jnp.int32)

    run = pl.kernel(
        _body,
        out_type=jax.ShapeDtypeStruct((BATCH, SEQ, D), jnp.float32),
        mesh=plsc.VectorSubcoreMesh(core_axis_name="c", subcore_axis_name="s"),
        compiler_params=pltpu.CompilerParams(use_tc_tiling_on_sc=False),
        scratch_types=(
            [pltpu.VMEM((QROWS, D), jnp.float32),
             pltpu.VMEM((RPT,), jnp.int32)]
            + [pltpu.VMEM((SEQ, D), jnp.float32)] * (NIB + NOB)
            + [pltpu.SemaphoreType.DMA] * (NIB + NOB)
        ),
    )
    return run(x, xl, q)
